# Initial kernel scaffold; baseline (speedup 1.0000x reference)
import jax
import jax.numpy as jnp
from jax.experimental import pallas as pl


def kernel(nfeat, edge_index, W_neigh, b_neigh):
    raise NotImplementedError("write your pallas kernel here")


# speedup vs baseline: 5.9828x; 5.9828x over previous
import jax
import jax.numpy as jnp
from jax import lax
from jax.experimental import pallas as pl
from jax.experimental.pallas import tpu as pltpu
from jax.experimental.pallas import tpu_sc as plsc

N_NODES = 10000
N_EDGES = 320000
D = 128

NC = 2
NS = 16
NW = NC * NS
EW = N_EDGES // NW
K = 80
CH = EW // K
NP = 10240
RPT = NP // NS
RCH = 128


def _sc_scatter_body(nfeat_hbm, src_hbm, dst_hbm,
                     acc_out, deg_out,
                     idx_s, idx_d, rows, deg_io, ones_v, sem,
                     acc_sh, deg_sh):
    c = lax.axis_index("c")
    s = lax.axis_index("s")
    wid = c * NS + s

    one16 = jnp.full((16,), 1.0, dtype=jnp.float32)
    zero16 = jnp.zeros((16,), dtype=jnp.float32)
    for i in range(K // 16):
        ones_v[pl.ds(i * 16, 16)] = one16
    for i in range(RCH // 16):
        deg_io[pl.ds(i * 16, 16)] = zero16

    for r in range(RPT // RCH):
        r0 = s * RPT + r * RCH
        pltpu.sync_copy(nfeat_hbm.at[pl.ds(r0, RCH)], rows)
        pltpu.sync_copy(rows, acc_sh.at[pl.ds(r0, RCH)])
        pltpu.sync_copy(deg_io, deg_sh.at[pl.ds(r0, RCH)])
    plsc.subcore_barrier()

    ebase = wid * EW

    def chunk(i, carry):
        b = pl.multiple_of(ebase + i * K, 8)
        pltpu.sync_copy(src_hbm.at[pl.ds(b, K)], idx_s)
        pltpu.sync_copy(dst_hbm.at[pl.ds(b, K)], idx_d)
        pltpu.async_copy(nfeat_hbm.at[idx_s], rows.at[pl.ds(0, K)], sem).wait()
        pltpu.sync_copy(rows.at[pl.ds(0, K)], acc_sh.at[idx_d], add=True)
        pltpu.sync_copy(ones_v, deg_sh.at[idx_d], add=True)
        return carry

    lax.fori_loop(0, CH, chunk, 0)
    plsc.subcore_barrier()

    for r in range(RPT // RCH):
        r0 = s * RPT + r * RCH
        pltpu.sync_copy(acc_sh.at[pl.ds(r0, RCH)], rows)
        pltpu.sync_copy(rows, acc_out.at[c].at[pl.ds(r0, RCH)])
        pltpu.sync_copy(deg_sh.at[pl.ds(r0, RCH)], deg_io)
        pltpu.sync_copy(deg_io, deg_out.at[c].at[pl.ds(r0, RCH)])


_sc_scatter = pl.kernel(
    _sc_scatter_body,
    out_type=[
        jax.ShapeDtypeStruct((NC, NP, D), jnp.float32),
        jax.ShapeDtypeStruct((NC, NP), jnp.float32),
    ],
    mesh=plsc.VectorSubcoreMesh(core_axis_name="c", subcore_axis_name="s",
                                num_cores=NC, num_subcores=NS),
    scratch_types=[
        pltpu.VMEM((K,), jnp.int32),
        pltpu.VMEM((K,), jnp.int32),
        pltpu.VMEM((RCH, D), jnp.float32),
        pltpu.VMEM((RCH,), jnp.float32),
        pltpu.VMEM((K,), jnp.float32),
        pltpu.SemaphoreType.DMA,
        pltpu.VMEM_SHARED((NP, D), jnp.float32),
        pltpu.VMEM_SHARED((NP,), jnp.float32),
    ],
)


def _tc_combine_body(acc_ref, deg_ref, wt_ref, b_ref, out_ref):
    a = acc_ref[0] + acc_ref[1]
    d = deg_ref[0] + deg_ref[1] + 1.0
    y = jnp.dot(a, wt_ref[...], preferred_element_type=jnp.float32)
    out_ref[...] = (y + b_ref[...]) / d + b_ref[...]


def _tc_combine(acc, deg, wt, b):
    blk = 1000
    grid = (N_NODES // blk,)
    return pl.pallas_call(
        _tc_combine_body,
        grid=grid,
        in_specs=[
            pl.BlockSpec((NC, blk, D), lambda i: (0, i, 0)),
            pl.BlockSpec((NC, blk, 1), lambda i: (0, i, 0)),
            pl.BlockSpec((D, D), lambda i: (0, 0)),
            pl.BlockSpec((1, D), lambda i: (0, 0)),
        ],
        out_specs=pl.BlockSpec((blk, D), lambda i: (i, 0)),
        out_shape=jax.ShapeDtypeStruct((N_NODES, D), jnp.float32),
    )(acc, deg, wt, b)


def kernel(nfeat, edge_index, W_neigh, b_neigh):
    src = edge_index[0].astype(jnp.int32)
    dst = edge_index[1].astype(jnp.int32)
    nfeat_p = jnp.pad(nfeat, ((0, NP - N_NODES), (0, 0)))
    acc, degf = _sc_scatter(nfeat_p, src, dst)
    acc = acc[:, :N_NODES]
    deg = degf[:, :N_NODES, None]
    return _tc_combine(acc, deg, W_neigh.T, b_neigh[None, :])


# speedup vs baseline: 9.0230x; 1.5081x over previous
import jax
import jax.numpy as jnp
from jax import lax
from jax.experimental import pallas as pl
from jax.experimental.pallas import tpu as pltpu
from jax.experimental.pallas import tpu_sc as plsc

N_NODES = 10000
N_EDGES = 320000
D = 128

NC = 2
NS = 16
NW = NC * NS
EW = N_EDGES // NW
K = 80
CH = EW // K
NP = 10240
RPT = NP // NS
RCH = 128


def _sc_scatter_body(nfeat_hbm, src_hbm, dst_hbm,
                     acc_out, deg_out,
                     idx_s, idx_d, rows, idx_s2, idx_d2, rows2,
                     deg_io, ones_v, sem, sem2,
                     acc_sh, deg_sh):
    c = lax.axis_index("c")
    s = lax.axis_index("s")
    wid = c * NS + s

    one16 = jnp.full((16,), 1.0, dtype=jnp.float32)
    zero16 = jnp.zeros((16,), dtype=jnp.float32)
    for i in range(K // 16):
        ones_v[pl.ds(i * 16, 16)] = one16
    for i in range(RCH // 16):
        deg_io[pl.ds(i * 16, 16)] = zero16

    for r in range(RPT // RCH):
        r0 = s * RPT + r * RCH
        pltpu.sync_copy(nfeat_hbm.at[pl.ds(r0, RCH)], rows)
        pltpu.sync_copy(rows, acc_sh.at[pl.ds(r0, RCH)])
        pltpu.sync_copy(deg_io, deg_sh.at[pl.ds(r0, RCH)])
    plsc.subcore_barrier()

    ebase = wid * EW

    def issue(i, idx_sx, idx_dx, rows_x, sem_x):
        b = pl.multiple_of(ebase + i * K, 8)
        pltpu.sync_copy(src_hbm.at[pl.ds(b, K)], idx_sx)
        pltpu.sync_copy(dst_hbm.at[pl.ds(b, K)], idx_dx)
        pltpu.async_copy(nfeat_hbm.at[idx_sx], rows_x, sem_x)

    def drain(idx_dx, rows_x, sem_x):
        pltpu.make_async_copy(nfeat_hbm.at[idx_sx_dummy], rows_x, sem_x).wait()
        pltpu.sync_copy(rows_x, acc_sh.at[idx_dx], add=True)
        pltpu.sync_copy(ones_v, deg_sh.at[idx_dx], add=True)

    idx_sx_dummy = idx_s
    rows_a = rows.at[pl.ds(0, K)]

    issue(0, idx_s, idx_d, rows_a, sem)

    def pair(g, carry):
        issue(2 * g + 1, idx_s2, idx_d2, rows2, sem2)
        drain(idx_d, rows_a, sem)

        @pl.when(2 * g + 2 < CH)
        def _():
            issue(2 * g + 2, idx_s, idx_d, rows_a, sem)

        drain(idx_d2, rows2, sem2)
        return carry

    lax.fori_loop(0, CH // 2, pair, 0)
    drain(idx_d, rows_a, sem)
    plsc.subcore_barrier()

    for r in range(RPT // RCH):
        r0 = s * RPT + r * RCH
        pltpu.sync_copy(acc_sh.at[pl.ds(r0, RCH)], rows)
        pltpu.sync_copy(rows, acc_out.at[c].at[pl.ds(r0, RCH)])
        pltpu.sync_copy(deg_sh.at[pl.ds(r0, RCH)], deg_io)
        pltpu.sync_copy(deg_io, deg_out.at[c].at[pl.ds(r0, RCH)])


_sc_scatter = pl.kernel(
    _sc_scatter_body,
    out_type=[
        jax.ShapeDtypeStruct((NC, NP, D), jnp.float32),
        jax.ShapeDtypeStruct((NC, NP), jnp.float32),
    ],
    mesh=plsc.VectorSubcoreMesh(core_axis_name="c", subcore_axis_name="s",
                                num_cores=NC, num_subcores=NS),
    scratch_types=[
        pltpu.VMEM((K,), jnp.int32),
        pltpu.VMEM((K,), jnp.int32),
        pltpu.VMEM((RCH, D), jnp.float32),
        pltpu.VMEM((K,), jnp.int32),
        pltpu.VMEM((K,), jnp.int32),
        pltpu.VMEM((K, D), jnp.float32),
        pltpu.VMEM((RCH,), jnp.float32),
        pltpu.VMEM((K,), jnp.float32),
        pltpu.SemaphoreType.DMA,
        pltpu.SemaphoreType.DMA,
        pltpu.VMEM_SHARED((NP, D), jnp.float32),
        pltpu.VMEM_SHARED((NP,), jnp.float32),
    ],
)


def _tc_combine_body(acc_ref, deg_ref, wt_ref, b_ref, out_ref):
    a = acc_ref[0] + acc_ref[1]
    d = deg_ref[0] + deg_ref[1] + 1.0
    y = jnp.dot(a, wt_ref[...], preferred_element_type=jnp.float32)
    out_ref[...] = (y + b_ref[...]) / d + b_ref[...]


def _tc_combine(acc, deg, wt, b):
    blk = 1000
    grid = (N_NODES // blk,)
    return pl.pallas_call(
        _tc_combine_body,
        grid=grid,
        in_specs=[
            pl.BlockSpec((NC, blk, D), lambda i: (0, i, 0)),
            pl.BlockSpec((NC, blk, 1), lambda i: (0, i, 0)),
            pl.BlockSpec((D, D), lambda i: (0, 0)),
            pl.BlockSpec((1, D), lambda i: (0, 0)),
        ],
        out_specs=pl.BlockSpec((blk, D), lambda i: (i, 0)),
        out_shape=jax.ShapeDtypeStruct((N_NODES, D), jnp.float32),
    )(acc, deg, wt, b)


def kernel(nfeat, edge_index, W_neigh, b_neigh):
    src = edge_index[0].astype(jnp.int32)
    dst = edge_index[1].astype(jnp.int32)
    nfeat_p = jnp.pad(nfeat, ((0, NP - N_NODES), (0, 0)))
    acc, degf = _sc_scatter(nfeat_p, src, dst)
    acc = acc[:, :N_NODES]
    deg = degf[:, :N_NODES, None]
    return _tc_combine(acc, deg, W_neigh.T, b_neigh[None, :])
